# baseline (speedup 1.0000x reference)
import functools

import jax
import jax.numpy as jnp
from jax.experimental import pallas as pl
from jax.experimental.pallas import tpu as pltpu

_EPS = 1e-8


def _body(w_ref, m_ref, keys_ref, vals_ref, out_ref, best_s_ref, best_i_ref,
          sem, *, rows):
    i = pl.program_id(0)

    @pl.when(i == 0)
    def _init():
        best_s_ref[0] = -jnp.inf
        best_i_ref[0] = 0

    b = keys_ref[...]
    dn = (((1,), (1,)), ((), ()))
    dots = jax.lax.dot_general(w_ref[...], b, dn,
                               preferred_element_type=jnp.float32)
    sumsq = jax.lax.dot_general(m_ref[...], b * b, dn,
                                preferred_element_type=jnp.float32)
    scores = dots / jnp.maximum(jnp.sqrt(sumsq), _EPS)
    gidx = jax.lax.broadcasted_iota(jnp.int32, scores.shape, 1) + i * rows
    local_max = jnp.max(scores)
    local_arg = jnp.min(jnp.where(scores == local_max, gidx,
                                  jnp.int32(2147483647)))

    @pl.when(local_max > best_s_ref[0])
    def _update():
        best_s_ref[0] = local_max
        best_i_ref[0] = local_arg

    @pl.when(i == pl.num_programs(0) - 1)
    def _gather():
        idx = best_i_ref[0]
        cp = pltpu.make_async_copy(vals_ref.at[pl.ds(idx, 1), :], out_ref, sem)
        cp.start()
        cp.wait()


def _pick_rows(n):
    for r in (10000, 8000, 6250, 5000, 4000, 2500, 2000,
              1250, 1000, 625, 500, 250, 200, 125, 100, 50, 25, 20, 10, 8, 5,
              4, 2, 1):
        if r <= n and n % r == 0:
            return r
    return 1


def kernel(key, keys, vals):
    n, d = keys.shape
    rows = _pick_rows(n)

    w = key.reshape(1, d).astype(jnp.float32)
    m = jnp.ones((1, d), jnp.float32)

    out = pl.pallas_call(
        functools.partial(_body, rows=rows),
        grid=(n // rows,),
        in_specs=[
            pl.BlockSpec((1, d), lambda i: (0, 0)),
            pl.BlockSpec((1, d), lambda i: (0, 0)),
            pl.BlockSpec((rows, d), lambda i: (i, 0)),
            pl.BlockSpec(memory_space=pltpu.MemorySpace.HBM),
        ],
        out_specs=pl.BlockSpec(memory_space=pltpu.MemorySpace.HBM),
        out_shape=jax.ShapeDtypeStruct((1, vals.shape[1]), jnp.float32),
        scratch_shapes=[
            pltpu.SMEM((1,), jnp.float32),
            pltpu.SMEM((1,), jnp.int32),
            pltpu.SemaphoreType.DMA,
        ],
    )(w, m, keys, vals)
    return out.reshape(vals.shape[1])


# speedup vs baseline: 1.0378x; 1.0378x over previous
import functools

import jax
import jax.numpy as jnp
from jax.experimental import pallas as pl
from jax.experimental.pallas import tpu as pltpu

_EPS = 1e-8
_PACK = 8


def _body(w_ref, m_ref, keys_ref, vals_ref, out_ref, buf, sems, gsem,
          *, n, rows, nsplit):
    num_chunks = n // rows
    sub = rows // nsplit

    def chunk_copies(c, slot):
        base = c * rows
        return [pltpu.make_async_copy(
            keys_ref.at[pl.ds(base + s * sub, sub), :],
            buf.at[slot, pl.ds(s * sub, sub), :],
            sems.at[slot, s]) for s in range(nsplit)]

    for cp in chunk_copies(0, 0):
        cp.start()

    w = w_ref[...]
    m = m_ref[...]
    dn = (((1,), (1,)), ((), ()))

    def loop(c, carry):
        best_s, best_i = carry
        slot = jax.lax.rem(c, 2)

        @pl.when(c + 1 < num_chunks)
        def _prefetch():
            for cp in chunk_copies(c + 1, 1 - slot):
                cp.start()

        for cp in chunk_copies(c, slot):
            cp.wait()
        b = buf[slot]
        dots = jax.lax.dot_general(w, b, dn,
                                   preferred_element_type=jnp.float32)
        sumsq = jax.lax.dot_general(m, b * b, dn,
                                    preferred_element_type=jnp.float32)
        scores = dots / jnp.maximum(jnp.sqrt(sumsq), _EPS)
        gidx = (jax.lax.broadcasted_iota(jnp.int32, scores.shape, 1)
                + c * rows)
        local_max = jnp.max(scores)
        local_arg = jnp.min(jnp.where(scores == local_max, gidx,
                                      jnp.int32(2147483647)))
        pick = local_max > best_s
        return (jnp.where(pick, local_max, best_s),
                jnp.where(pick, local_arg, best_i))

    _, best_i = jax.lax.fori_loop(
        0, num_chunks, loop, (jnp.float32(-jnp.inf), jnp.int32(0)))

    cp = pltpu.make_async_copy(vals_ref.at[pl.ds(best_i, 1), :], out_ref, gsem)
    cp.start()
    cp.wait()


def _pick_rows(n):
    for r in (20000, 16000, 10000, 8000, 5000, 4000, 2000, 1000, 500, 200,
              100, 40, 20, 8, 4, 2, 1):
        if r <= n and n % r == 0:
            return r
    return 1


def kernel(key, keys, vals):
    n, d = keys.shape
    rows = _pick_rows(n)
    nsplit = 8 if rows % 8 == 0 else 1

    w = key.reshape(1, d).astype(jnp.float32)
    m = jnp.ones((1, d), jnp.float32)

    out = pl.pallas_call(
        functools.partial(_body, n=n, rows=rows, nsplit=nsplit),
        in_specs=[
            pl.BlockSpec(memory_space=pltpu.MemorySpace.VMEM),
            pl.BlockSpec(memory_space=pltpu.MemorySpace.VMEM),
            pl.BlockSpec(memory_space=pltpu.MemorySpace.HBM),
            pl.BlockSpec(memory_space=pltpu.MemorySpace.HBM),
        ],
        out_specs=pl.BlockSpec(memory_space=pltpu.MemorySpace.HBM),
        out_shape=jax.ShapeDtypeStruct((1, vals.shape[1]), jnp.float32),
        scratch_shapes=[
            pltpu.VMEM((2, rows, d), jnp.float32),
            pltpu.SemaphoreType.DMA((2, 8)),
            pltpu.SemaphoreType.DMA,
        ],
    )(w, m, keys, vals)
    return out.reshape(vals.shape[1])


# speedup vs baseline: 2.1271x; 2.0496x over previous
import functools

import jax
import jax.numpy as jnp
from jax.experimental import pallas as pl
from jax.experimental.pallas import tpu as pltpu

_EPS = 1e-8


def _body(w_ref, m_ref, kt_ref, vals_ref, out_ref, best_s_ref, best_i_ref,
          sem, *, lanes, n):
    i = pl.program_id(0)

    @pl.when(i == 0)
    def _init():
        best_s_ref[0] = -jnp.inf
        best_i_ref[0] = 0

    b = kt_ref[...]
    dn = (((1,), (0,)), ((), ()))
    dots = jax.lax.dot_general(w_ref[...], b, dn,
                               preferred_element_type=jnp.float32)
    sumsq = jax.lax.dot_general(m_ref[...], b * b, dn,
                                preferred_element_type=jnp.float32)
    scores = dots / jnp.maximum(jnp.sqrt(sumsq), _EPS)
    gidx = jax.lax.broadcasted_iota(jnp.int32, scores.shape, 1) + i * lanes
    scores = jnp.where(gidx < n, scores, -jnp.inf)
    local_max = jnp.max(scores)
    local_arg = jnp.min(jnp.where(scores == local_max, gidx,
                                  jnp.int32(2147483647)))

    @pl.when(local_max > best_s_ref[0])
    def _update():
        best_s_ref[0] = local_max
        best_i_ref[0] = local_arg

    @pl.when(i == pl.num_programs(0) - 1)
    def _gather():
        idx = best_i_ref[0]
        cp = pltpu.make_async_copy(vals_ref.at[pl.ds(idx, 1), :], out_ref, sem)
        cp.start()
        cp.wait()


def _pick_lanes(n):
    if n <= 65536:
        return n
    return 64000


def kernel(key, keys, vals):
    n, d = keys.shape
    lanes = _pick_lanes(n)
    kt = keys.T

    w = key.reshape(1, d).astype(jnp.float32)
    m = jnp.ones((1, d), jnp.float32)

    out = pl.pallas_call(
        functools.partial(_body, lanes=lanes, n=n),
        grid=((n + lanes - 1) // lanes,),
        in_specs=[
            pl.BlockSpec((1, d), lambda i: (0, 0)),
            pl.BlockSpec((1, d), lambda i: (0, 0)),
            pl.BlockSpec((d, lanes), lambda i: (0, i)),
            pl.BlockSpec(memory_space=pltpu.MemorySpace.HBM),
        ],
        out_specs=pl.BlockSpec(memory_space=pltpu.MemorySpace.HBM),
        out_shape=jax.ShapeDtypeStruct((1, vals.shape[1]), jnp.float32),
        scratch_shapes=[
            pltpu.SMEM((1,), jnp.float32),
            pltpu.SMEM((1,), jnp.int32),
            pltpu.SemaphoreType.DMA,
        ],
    )(w, m, kt, vals)
    return out.reshape(vals.shape[1])


# speedup vs baseline: 2.1304x; 1.0016x over previous
import functools

import jax
import jax.numpy as jnp
from jax.experimental import pallas as pl
from jax.experimental.pallas import tpu as pltpu

_EPS = 1e-8
_NSTREAM = 4


def _body(w_ref, m_ref, k0, k1, k2, k3, vals_ref, out_ref,
          best_s_ref, best_i_ref, sem, *, sub, n):
    i = pl.program_id(0)

    @pl.when(i == 0)
    def _init():
        best_s_ref[0] = -jnp.inf
        best_i_ref[0] = 0

    w = w_ref[...]
    m = m_ref[...]
    dn = (((1,), (0,)), ((), ()))

    bs = jnp.float32(-jnp.inf)
    bi = jnp.int32(0)
    for s, kr in enumerate((k0, k1, k2, k3)):
        b = kr[...]
        dots = jax.lax.dot_general(w, b, dn,
                                   preferred_element_type=jnp.float32)
        sumsq = jax.lax.dot_general(m, b * b, dn,
                                    preferred_element_type=jnp.float32)
        scores = dots / jnp.maximum(jnp.sqrt(sumsq), _EPS)
        gidx = (jax.lax.broadcasted_iota(jnp.int32, scores.shape, 1)
                + (i * _NSTREAM + s) * sub)
        scores = jnp.where(gidx < n, scores, -jnp.inf)
        lm = jnp.max(scores)
        la = jnp.min(jnp.where(scores == lm, gidx, jnp.int32(2147483647)))
        pick = lm > bs
        bs = jnp.where(pick, lm, bs)
        bi = jnp.where(pick, la, bi)

    @pl.when(bs > best_s_ref[0])
    def _update():
        best_s_ref[0] = bs
        best_i_ref[0] = bi

    @pl.when(i == pl.num_programs(0) - 1)
    def _gather():
        idx = best_i_ref[0]
        cp = pltpu.make_async_copy(vals_ref.at[pl.ds(idx, 1), :], out_ref, sem)
        cp.start()
        cp.wait()


def kernel(key, keys, vals):
    n, d = keys.shape
    kt = keys.T
    if n >= 1024 * _NSTREAM:
        sub = 16000 if n >= 16000 * _NSTREAM else 128 * (n // (128 * _NSTREAM))
    else:
        sub = None

    w = key.reshape(1, d).astype(jnp.float32)
    m = jnp.ones((1, d), jnp.float32)

    if sub is None:
        sub_eff = n
        grid = (1,)
        kspecs = [pl.BlockSpec((d, n), lambda i: (0, 0))] * _NSTREAM
        sub = 0
        body = functools.partial(_body, sub=0, n=n)
    else:
        step = _NSTREAM * sub
        grid = ((n + step - 1) // step,)
        max_block = (n + sub - 1) // sub - 1

        def mk(s):
            return pl.BlockSpec(
                (d, sub),
                lambda i, s=s: (0, jnp.minimum(i * _NSTREAM + s, max_block)))

        kspecs = [mk(s) for s in range(_NSTREAM)]
        body = functools.partial(_body, sub=sub, n=n)

    out = pl.pallas_call(
        body,
        grid=grid,
        in_specs=[
            pl.BlockSpec((1, d), lambda i: (0, 0)),
            pl.BlockSpec((1, d), lambda i: (0, 0)),
            *kspecs,
            pl.BlockSpec(memory_space=pltpu.MemorySpace.HBM),
        ],
        out_specs=pl.BlockSpec(memory_space=pltpu.MemorySpace.HBM),
        out_shape=jax.ShapeDtypeStruct((1, vals.shape[1]), jnp.float32),
        scratch_shapes=[
            pltpu.SMEM((1,), jnp.float32),
            pltpu.SMEM((1,), jnp.int32),
            pltpu.SemaphoreType.DMA,
        ],
    )(w, m, kt, kt, kt, kt, vals)
    return out.reshape(vals.shape[1])
